# baseline (speedup 1.0000x reference)
import functools

import jax
import jax.numpy as jnp
from jax import lax
from jax.experimental import pallas as pl
from jax.experimental.pallas import tpu as pltpu, tpu_sc as plsc

_VEC = 16
_SUB = 8


@functools.lru_cache(maxsize=None)
def _make_gather(V, D, B):
    info = plsc.get_sparse_core_info()
    nw = info.num_cores * info.num_subcores
    b_per_w = B // nw
    n_vecs = b_per_w // _VEC
    half = V // 2
    mesh = plsc.VectorSubcoreMesh(core_axis_name="c", subcore_axis_name="s")

    @functools.partial(
        pl.kernel,
        mesh=mesh,
        compiler_params=pltpu.CompilerParams(
            use_tc_tiling_on_sc=True, needs_layout_passes=False
        ),
        out_type=jax.ShapeDtypeStruct((B, D), jnp.float32),
        scratch_types=[
            pltpu.VMEM((b_per_w,), jnp.int32),
            pltpu.VMEM((b_per_w, D), jnp.float32),
            pltpu.SemaphoreType.DMA,
        ],
    )
    def k(ta_hbm, tb_hbm, idx_hbm, out_hbm, idx_v, rows_v, sem):
        wid = lax.axis_index("s") * info.num_cores + lax.axis_index("c")
        base = wid * b_per_w
        pltpu.sync_copy(idx_hbm.at[pl.ds(base, b_per_w)], idx_v)
        lanes = lax.iota(jnp.int32, _VEC)

        def body(v, carry):
            vec = idx_v[pl.ds(v * _VEC, _VEC)]
            for i in range(_VEC):
                r = jnp.sum(jnp.where(lanes == i, vec, 0))
                in_a = r < half
                rr = jnp.where(in_a, r, r - half)
                t = lax.shift_right_logical(rr, 3)
                s = lax.bitwise_and(rr, _SUB - 1)
                dst = rows_v.at[v * _VEC + i]

                @pl.when(in_a)
                def _():
                    pltpu.async_copy(ta_hbm.at[t, s], dst, sem)

                @pl.when(jnp.logical_not(in_a))
                def _():
                    pltpu.async_copy(tb_hbm.at[t, s], dst, sem)

            return carry

        lax.fori_loop(0, n_vecs, body, 0)
        pltpu.make_async_copy(out_hbm.at[pl.ds(base, b_per_w)], rows_v, sem).wait()
        pltpu.sync_copy(rows_v, out_hbm.at[pl.ds(base, b_per_w)])

    return k


@jax.jit
def kernel(source, hidden, cell, emb):
    V, D = emb.shape
    B = source.shape[0]
    ta = emb[: V // 2].reshape(V // 16, _SUB, D)
    tb = emb[V // 2 :].reshape(V // 16, _SUB, D)
    return _make_gather(V, D, B)(ta, tb, source)


# speedup vs baseline: 2.1781x; 2.1781x over previous
import functools

import jax
import jax.numpy as jnp
from jax import lax
from jax.experimental import pallas as pl
from jax.experimental.pallas import tpu as pltpu, tpu_sc as plsc

_VEC = 16
_LANE = 128
_G = 2


@functools.lru_cache(maxsize=None)
def _make_gather(V, D, B):
    info = plsc.get_sparse_core_info()
    nw = info.num_cores * info.num_subcores
    b_per_w = B // nw
    n_steps = b_per_w // _G
    mesh = plsc.VectorSubcoreMesh(core_axis_name="c", subcore_axis_name="s")

    @functools.partial(
        pl.kernel,
        mesh=mesh,
        compiler_params=pltpu.CompilerParams(
            use_tc_tiling_on_sc=True, needs_layout_passes=False
        ),
        out_type=jax.ShapeDtypeStruct((B, D), jnp.float32),
        scratch_types=[
            pltpu.VMEM((b_per_w,), jnp.int32),
            pltpu.VMEM((2, _G, D // 8, 8, _LANE), jnp.float32),
            pltpu.VMEM((b_per_w, D), jnp.float32),
            pltpu.SemaphoreType.DMA,
        ],
    )
    def k(table_hbm, idx_hbm, out_hbm, idx_v, slabs_v, rows_v, sem):
        wid = lax.axis_index("s") * info.num_cores + lax.axis_index("c")
        base = wid * b_per_w
        pltpu.sync_copy(idx_hbm.at[pl.ds(base, b_per_w)], idx_v)
        lanes = lax.iota(jnp.int32, _VEC)

        per_vec = _VEC // _G
        pv_shift = per_vec.bit_length() - 1

        def extract(c, g):
            vec = idx_v[
                pl.ds(lax.shift_left(lax.shift_right_logical(c, pv_shift), 4), _VEC)
            ]
            lane_id = lax.bitwise_and(c, per_vec - 1) * _G + g
            return jnp.sum(jnp.where(lanes == lane_id, vec, 0))

        def fire(c):
            par = lax.bitwise_and(c, 1)
            for g in range(_G):
                r = extract(c, g)
                blk = lax.shift_right_logical(r, 7) * _LANE
                for t in range(D // 8):
                    pltpu.async_copy(
                        table_hbm.at[pl.ds(t * 8, 8), pl.ds(blk, _LANE)],
                        slabs_v.at[par, g, t],
                        sem,
                    )

        def step(c, carry):
            @pl.when(c + 1 < n_steps)
            def _():
                fire(c + 1)

            par = lax.bitwise_and(c, 1)
            for g in range(_G):
                for t in range(D // 8):
                    pltpu.make_async_copy(
                        table_hbm.at[pl.ds(0, 8), pl.ds(0, _LANE)],
                        slabs_v.at[par, g, t],
                        sem,
                    ).wait()
            pv = jnp.full((_VEC,), par, jnp.int32)
            for g in range(_G):
                r = extract(c, g)
                lane = jnp.full((_VEC,), lax.bitwise_and(r, _LANE - 1))
                gv = jnp.full((_VEC,), g, jnp.int32)
                j = c * _G + g
                for f in range(D // _VEC):
                    feats = lanes + f * _VEC
                    t_vec = lax.shift_right_logical(feats, 3)
                    s_vec = lax.bitwise_and(feats, 7)
                    vals = plsc.load_gather(
                        slabs_v, [pv, gv, t_vec, s_vec, lane]
                    )
                    rows_v[j, pl.ds(f * _VEC, _VEC)] = vals
            return carry

        fire(0)
        lax.fori_loop(0, n_steps, step, 0)
        pltpu.sync_copy(rows_v, out_hbm.at[pl.ds(base, b_per_w)])

    return k


@jax.jit
def kernel(source, hidden, cell, emb):
    V, D = emb.shape
    B = source.shape[0]
    return _make_gather(V, D, B)(emb.T, source)


# speedup vs baseline: 2.5433x; 1.1677x over previous
import functools

import jax
import jax.numpy as jnp
from jax import lax
from jax.experimental import pallas as pl
from jax.experimental.pallas import tpu as pltpu, tpu_sc as plsc

_VEC = 16
_LANE = 128
_G = 4
_RCHUNK = 64


@functools.lru_cache(maxsize=None)
def _make_gather(V, D, B):
    info = plsc.get_sparse_core_info()
    nw = info.num_cores * info.num_subcores
    b_per_w = B // nw
    n_steps = b_per_w // _G
    mesh = plsc.VectorSubcoreMesh(core_axis_name="c", subcore_axis_name="s")

    @functools.partial(
        pl.kernel,
        mesh=mesh,
        compiler_params=pltpu.CompilerParams(
            use_tc_tiling_on_sc=True, needs_layout_passes=False
        ),
        out_type=jax.ShapeDtypeStruct((B, D), jnp.float32),
        scratch_types=[
            pltpu.VMEM((b_per_w,), jnp.int32),
            pltpu.VMEM((2, _G, D // 8, 8, _LANE), jnp.float32),
            pltpu.VMEM((_RCHUNK, D), jnp.float32),
            pltpu.SemaphoreType.DMA,
        ],
    )
    def k(table_hbm, idx_hbm, out_hbm, idx_v, slabs_v, rows_v, sem):
        wid = lax.axis_index("s") * info.num_cores + lax.axis_index("c")
        base = wid * b_per_w
        pltpu.sync_copy(idx_hbm.at[pl.ds(base, b_per_w)], idx_v)
        lanes = lax.iota(jnp.int32, _VEC)

        per_vec = _VEC // _G
        pv_shift = per_vec.bit_length() - 1

        def extract(c, g):
            vec = idx_v[
                pl.ds(lax.shift_left(lax.shift_right_logical(c, pv_shift), 4), _VEC)
            ]
            lane_id = lax.bitwise_and(c, per_vec - 1) * _G + g
            return jnp.sum(jnp.where(lanes == lane_id, vec, 0))

        def fire(c):
            par = lax.bitwise_and(c, 1)
            for g in range(_G):
                r = extract(c, g)
                blk = lax.shift_right_logical(r, 7) * _LANE
                for t in range(D // 8):
                    pltpu.async_copy(
                        table_hbm.at[pl.ds(t * 8, 8), pl.ds(blk, _LANE)],
                        slabs_v.at[par, g, t],
                        sem,
                    )

        def step(c, carry):
            @pl.when(c + 1 < n_steps)
            def _():
                fire(c + 1)

            par = lax.bitwise_and(c, 1)
            for g in range(_G):
                for t in range(D // 8):
                    pltpu.make_async_copy(
                        table_hbm.at[pl.ds(0, 8), pl.ds(0, _LANE)],
                        slabs_v.at[par, g, t],
                        sem,
                    ).wait()
            pv = jnp.full((_VEC,), par, jnp.int32)
            steps_per_chunk = _RCHUNK // _G
            jc = lax.rem(c, steps_per_chunk)
            for g in range(_G):
                r = extract(c, g)
                lane = jnp.full((_VEC,), lax.bitwise_and(r, _LANE - 1))
                gv = jnp.full((_VEC,), g, jnp.int32)
                j = jc * _G + g
                for f in range(D // _VEC):
                    feats = lanes + f * _VEC
                    t_vec = lax.shift_right_logical(feats, 3)
                    s_vec = lax.bitwise_and(feats, 7)
                    vals = plsc.load_gather(
                        slabs_v, [pv, gv, t_vec, s_vec, lane]
                    )
                    rows_v[j, pl.ds(f * _VEC, _VEC)] = vals

            @pl.when(jc == steps_per_chunk - 1)
            def _():
                chunk = lax.div(c, steps_per_chunk)
                pltpu.sync_copy(
                    rows_v, out_hbm.at[pl.ds(base + chunk * _RCHUNK, _RCHUNK)]
                )

            return carry

        fire(0)
        lax.fori_loop(0, n_steps, step, 0)

    return k


@jax.jit
def kernel(source, hidden, cell, emb):
    V, D = emb.shape
    B = source.shape[0]
    return _make_gather(V, D, B)(emb.T, source)
